# baseline (speedup 1.0000x reference)
import functools

import jax
import jax.numpy as jnp
from jax import lax
from jax.experimental import pallas as pl
from jax.experimental.pallas import tpu as pltpu
from jax.experimental.pallas import tpu_sc as plsc

DT = 128


def _make_sc_gather(MP, n_tab):
    info = plsc.get_sparse_core_info()
    NC, NS = info.num_cores, info.num_subcores
    NW = NC * NS
    R = (MP * 27) // NW
    CH = 120
    NCH = R // CH
    assert (MP * 27) % NW == 0 and R % CH == 0 and NCH % 2 == 0

    mesh = plsc.VectorSubcoreMesh(core_axis_name="c", subcore_axis_name="s")
    out_type = tuple(jax.ShapeDtypeStruct((MP * 27, DT), jnp.float32)
                     for _ in range(n_tab))
    if n_tab == 1:
        out_type = out_type[0]
    scratch = [pltpu.VMEM((R,), jnp.int32)]
    for _ in range(n_tab):
        scratch.append(pltpu.VMEM((CH, DT), jnp.float32))
        scratch.append(pltpu.VMEM((CH, DT), jnp.float32))
    scratch.append(pltpu.SemaphoreType.DMA)
    scratch.append(pltpu.SemaphoreType.DMA)

    @functools.partial(pl.kernel, mesh=mesh, out_type=out_type,
                       scratch_types=tuple(scratch))
    def gather_kernel(*refs):
        tables = refs[:n_tab]
        idx_hbm = refs[n_tab]
        outs = refs[n_tab + 1: 2 * n_tab + 1]
        idx_v = refs[2 * n_tab + 1]
        bufs = refs[2 * n_tab + 2: 2 * n_tab + 2 + 2 * n_tab]
        s0 = refs[-2]
        s1 = refs[-1]

        wid = lax.axis_index("s") * NC + lax.axis_index("c")
        base = wid * R
        pltpu.sync_copy(idx_hbm.at[pl.ds(base, R)], idx_v)

        for t in range(n_tab):
            table = tables[t]
            out = outs[t]
            b0 = bufs[2 * t]
            b1 = bufs[2 * t + 1]

            def start(c, buf, sem):
                pltpu.async_copy(table.at[idx_v.at[pl.ds(c * CH, CH)]], buf, sem)

            def wait(buf, sem):
                pltpu.make_async_copy(table.at[pl.ds(0, CH)], buf, sem).wait()

            def write(c, buf):
                pltpu.sync_copy(buf, out.at[pl.ds(base + c * CH, CH)])

            start(0, b0, s0)

            def body(i, carry):
                c = 2 * i
                start(c + 1, b1, s1)
                wait(b0, s0)
                write(c, b0)
                start(c + 2, b0, s0)
                wait(b1, s1)
                write(c + 1, b1)
                return carry

            lax.fori_loop(0, NCH // 2 - 1, body, 0)
            c_last = NCH - 2
            start(c_last + 1, b1, s1)
            wait(b0, s0)
            write(c_last, b0)
            wait(b1, s1)
            write(c_last + 1, b1)

    return gather_kernel


def _row_mask(blk_m, n_real):
    def f(x):
        row = pl.program_id(0) * blk_m + lax.broadcasted_iota(jnp.int32, (blk_m, 1), 0)
        return jnp.where(row < n_real, x, 0.0)
    return f


def _widen(h, blk_m):
    return jnp.concatenate([h, jnp.zeros((blk_m, DT - 64), jnp.float32)], axis=1)


def _tc1(MP, n_real, blk_m=512):
    nblk = MP // blk_m
    maskf = _row_mask(blk_m, n_real)

    def body(g1, gq, w1, b1, wq1, bq1, wq2, bq2, wq3, bq3, wq4, bq4,
             h1_ref, c4_ref):
        h = jnp.dot(g1[...], w1[...], preferred_element_type=jnp.float32) + b1[...]
        h1_ref[...] = _widen(maskf(jnp.maximum(h, 0.0)), blk_m)
        c = jnp.dot(gq[...], wq1[...], preferred_element_type=jnp.float32) + bq1[...]
        c = jnp.maximum(c, 0.0)
        c = jnp.maximum(jnp.dot(c, wq2[...], preferred_element_type=jnp.float32) + bq2[...], 0.0)
        c = jnp.maximum(jnp.dot(c, wq3[...], preferred_element_type=jnp.float32) + bq3[...], 0.0)
        c4_ref[...] = jnp.dot(c, wq4[...], preferred_element_type=jnp.float32) + bq4[...]

    return pl.pallas_call(
        body,
        grid=(nblk,),
        in_specs=[
            pl.BlockSpec((blk_m, 27 * DT), lambda i: (i, 0)),
            pl.BlockSpec((blk_m, 27 * DT), lambda i: (i, 0)),
            pl.BlockSpec((27 * DT, 64), lambda i: (0, 0)),
            pl.BlockSpec((1, 64), lambda i: (0, 0)),
            pl.BlockSpec((27 * DT, 32), lambda i: (0, 0)),
            pl.BlockSpec((1, 32), lambda i: (0, 0)),
            pl.BlockSpec((32, 64), lambda i: (0, 0)),
            pl.BlockSpec((1, 64), lambda i: (0, 0)),
            pl.BlockSpec((64, 128), lambda i: (0, 0)),
            pl.BlockSpec((1, 128), lambda i: (0, 0)),
            pl.BlockSpec((128, 128), lambda i: (0, 0)),
            pl.BlockSpec((1, 128), lambda i: (0, 0)),
        ],
        out_specs=[
            pl.BlockSpec((blk_m, DT), lambda i: (i, 0)),
            pl.BlockSpec((blk_m, 128), lambda i: (i, 0)),
        ],
        out_shape=[
            jax.ShapeDtypeStruct((MP, DT), jnp.float32),
            jax.ShapeDtypeStruct((MP, 128), jnp.float32),
        ],
    )


def _tc_conv(MP, n_real, mode, blk_m=512):
    nblk = MP // blk_m
    maskf = _row_mask(blk_m, n_real)

    def body(g, w, b, extra, out_ref):
        h = jnp.dot(g[...], w[...], preferred_element_type=jnp.float32) + b[...]
        if mode == "film":
            e = extra[...]
            h = maskf(h * e[:, :64] + e[:, 64:])
            out_ref[...] = _widen(h, blk_m)
        elif mode == "relu":
            h = maskf(jnp.maximum(h, 0.0))
            out_ref[...] = _widen(h, blk_m)
        else:
            out_ref[...] = h + extra[...]

    extra_cols = 128 if mode == "film" else 64
    out_cols = 64 if mode == "residual" else DT
    return pl.pallas_call(
        body,
        grid=(nblk,),
        in_specs=[
            pl.BlockSpec((blk_m, 27 * DT), lambda i: (i, 0)),
            pl.BlockSpec((27 * DT, 64), lambda i: (0, 0)),
            pl.BlockSpec((1, 64), lambda i: (0, 0)),
            pl.BlockSpec((blk_m, extra_cols), lambda i: (i, 0)),
        ],
        out_specs=pl.BlockSpec((blk_m, out_cols), lambda i: (i, 0)),
        out_shape=jax.ShapeDtypeStruct((MP, out_cols), jnp.float32),
    )


def _pad_w(W, cin):
    cout = W.shape[2]
    return jnp.zeros((27, DT, cout), jnp.float32).at[:, :cin, :].set(W).reshape(27 * DT, cout)


def kernel(x_feats, cond_feats, nbr_idx, nbr_mask,
           W1a, b1a, W1b, b1b, W2a, b2a, W2b, b2b,
           Wq1, bq1, Wq2, bq2, Wq3, bq3, Wq4, bq4):
    n, N = x_feats.shape
    NQ = cond_feats.shape[1]
    MP = 10240
    assert n <= MP - 1

    xp = jnp.zeros((MP, DT), jnp.float32).at[:n, :N].set(x_feats)
    cp = jnp.zeros((MP, DT), jnp.float32).at[:n, :NQ].set(cond_feats)
    safe = jnp.where(nbr_mask, nbr_idx, n).astype(jnp.int32)
    idxflat = jnp.full((MP, 27), n, jnp.int32).at[:n].set(safe.T).reshape(MP * 27)

    w1af = _pad_w(W1a, N)
    w1bf = _pad_w(W1b, N)
    w2af = _pad_w(W2a, N)
    w2bf = _pad_w(W2b, N)
    wq1f = _pad_w(Wq1, NQ)

    def r2(b):
        return b.reshape(1, -1)

    gather2 = _make_sc_gather(MP, 2)
    gather1 = _make_sc_gather(MP, 1)
    tc1 = _tc1(MP, n)
    tc_film = _tc_conv(MP, n, "film")
    tc_relu = _tc_conv(MP, n, "relu")
    tc_res = _tc_conv(MP, n, "residual")

    G1, Gq = gather2(xp, cp, idxflat)
    h1, c4 = tc1(G1.reshape(MP, 27 * DT), Gq.reshape(MP, 27 * DT),
                 w1af, r2(b1a), wq1f, r2(bq1), Wq2, r2(bq2),
                 Wq3, r2(bq3), Wq4, r2(bq4))
    G2 = gather1(h1, idxflat)
    feats = tc_film(G2.reshape(MP, 27 * DT), w1bf, r2(b1b), c4)
    G3 = gather1(feats, idxflat)
    h2 = tc_relu(G3.reshape(MP, 27 * DT), w2af, r2(b2a), h1[:, :64])
    G4 = gather1(h2, idxflat)
    outp = tc_res(G4.reshape(MP, 27 * DT), w2bf, r2(b2b), xp[:, :64])
    return outp[:n]


# speedup vs baseline: 2.2117x; 2.2117x over previous
import functools

import jax
import jax.numpy as jnp
from jax import lax
from jax.experimental import pallas as pl
from jax.experimental.pallas import tpu as pltpu
from jax.experimental.pallas import tpu_sc as plsc

NB = 8


def _make_sc_gather(MP, dims):
    info = plsc.get_sparse_core_info()
    NC, NS = info.num_cores, info.num_subcores
    NW = NC * NS
    R = (MP * 27) // NW
    CH = 120
    NCH = R // CH
    assert (MP * 27) % NW == 0 and R % CH == 0 and NCH % NB == 0

    mesh = plsc.VectorSubcoreMesh(core_axis_name="c", subcore_axis_name="s")
    out_type = tuple(jax.ShapeDtypeStruct((MP * 27, D), jnp.float32) for D in dims)
    if len(dims) == 1:
        out_type = out_type[0]
    scratch = [pltpu.VMEM((R,), jnp.int32)]
    for D in dims:
        scratch.extend(pltpu.VMEM((CH, D), jnp.float32) for _ in range(NB))
    scratch.extend(pltpu.SemaphoreType.DMA for _ in range(NB))

    @functools.partial(pl.kernel, mesh=mesh, out_type=out_type,
                       scratch_types=tuple(scratch),
                       compiler_params=pltpu.CompilerParams(
                           use_tc_tiling_on_sc=False))
    def gather_kernel(*refs):
        nt = len(dims)
        tables = refs[:nt]
        idx_hbm = refs[nt]
        outs = refs[nt + 1: 2 * nt + 1]
        idx_v = refs[2 * nt + 1]
        allbufs = refs[2 * nt + 2: 2 * nt + 2 + NB * nt]
        sems = refs[2 * nt + 2 + NB * nt:]

        wid = lax.axis_index("s") * NC + lax.axis_index("c")
        base = wid * R
        pltpu.sync_copy(idx_hbm.at[pl.ds(base, R)], idx_v)

        for t in range(nt):
            table = tables[t]
            out = outs[t]
            bufs = allbufs[NB * t: NB * t + NB]

            def start(c, buf, sem):
                pltpu.async_copy(table.at[idx_v.at[pl.ds(c * CH, CH)]], buf, sem)

            def wait(buf, sem):
                pltpu.make_async_copy(table.at[pl.ds(0, CH)], buf, sem).wait()

            def write(c, buf):
                pltpu.sync_copy(buf, out.at[pl.ds(base + c * CH, CH)])

            for j in range(NB):
                start(j, bufs[j], sems[j])

            def body(g, carry):
                for j in range(NB):
                    c = g * NB + j
                    wait(bufs[j], sems[j])
                    write(c, bufs[j])
                    start(c + NB, bufs[j], sems[j])
                return carry

            lax.fori_loop(0, NCH // NB - 1, body, 0)
            for j in range(NB):
                c = (NCH // NB - 1) * NB + j
                wait(bufs[j], sems[j])
                write(c, bufs[j])

    return gather_kernel


def _row_mask(blk_m, n_real):
    def f(x):
        row = pl.program_id(0) * blk_m + lax.broadcasted_iota(jnp.int32, (blk_m, 1), 0)
        return jnp.where(row < n_real, x, 0.0)
    return f


def _tc1(MP, n_real, blk_m=512):
    nblk = MP // blk_m
    maskf = _row_mask(blk_m, n_real)

    def body(g1, gq, w1, b1, wq1, bq1, wq2, bq2, wq3, bq3, wq4, bq4,
             h1_ref, c4_ref):
        h = jnp.dot(g1[...], w1[...], preferred_element_type=jnp.float32) + b1[...]
        h1_ref[...] = maskf(jnp.maximum(h, 0.0))
        c = jnp.dot(gq[...], wq1[...], preferred_element_type=jnp.float32) + bq1[...]
        c = jnp.maximum(c, 0.0)
        c = jnp.maximum(jnp.dot(c, wq2[...], preferred_element_type=jnp.float32) + bq2[...], 0.0)
        c = jnp.maximum(jnp.dot(c, wq3[...], preferred_element_type=jnp.float32) + bq3[...], 0.0)
        c4_ref[...] = jnp.dot(c, wq4[...], preferred_element_type=jnp.float32) + bq4[...]

    return pl.pallas_call(
        body,
        grid=(nblk,),
        in_specs=[
            pl.BlockSpec((blk_m, 27 * 64), lambda i: (i, 0)),
            pl.BlockSpec((blk_m, 27 * 32), lambda i: (i, 0)),
            pl.BlockSpec((27 * 64, 64), lambda i: (0, 0)),
            pl.BlockSpec((1, 64), lambda i: (0, 0)),
            pl.BlockSpec((27 * 32, 32), lambda i: (0, 0)),
            pl.BlockSpec((1, 32), lambda i: (0, 0)),
            pl.BlockSpec((32, 64), lambda i: (0, 0)),
            pl.BlockSpec((1, 64), lambda i: (0, 0)),
            pl.BlockSpec((64, 128), lambda i: (0, 0)),
            pl.BlockSpec((1, 128), lambda i: (0, 0)),
            pl.BlockSpec((128, 128), lambda i: (0, 0)),
            pl.BlockSpec((1, 128), lambda i: (0, 0)),
        ],
        out_specs=[
            pl.BlockSpec((blk_m, 64), lambda i: (i, 0)),
            pl.BlockSpec((blk_m, 128), lambda i: (i, 0)),
        ],
        out_shape=[
            jax.ShapeDtypeStruct((MP, 64), jnp.float32),
            jax.ShapeDtypeStruct((MP, 128), jnp.float32),
        ],
    )


def _tc_conv(MP, n_real, mode, blk_m=512):
    nblk = MP // blk_m
    maskf = _row_mask(blk_m, n_real)

    def body(g, w, b, extra, out_ref):
        h = jnp.dot(g[...], w[...], preferred_element_type=jnp.float32) + b[...]
        if mode == "film":
            e = extra[...]
            h = maskf(h * e[:, :64] + e[:, 64:])
        elif mode == "relu":
            h = maskf(jnp.maximum(h, 0.0))
        else:
            h = h + extra[...]
        out_ref[...] = h

    extra_cols = 128 if mode == "film" else 64
    return pl.pallas_call(
        body,
        grid=(nblk,),
        in_specs=[
            pl.BlockSpec((blk_m, 27 * 64), lambda i: (i, 0)),
            pl.BlockSpec((27 * 64, 64), lambda i: (0, 0)),
            pl.BlockSpec((1, 64), lambda i: (0, 0)),
            pl.BlockSpec((blk_m, extra_cols), lambda i: (i, 0)),
        ],
        out_specs=pl.BlockSpec((blk_m, 64), lambda i: (i, 0)),
        out_shape=jax.ShapeDtypeStruct((MP, 64), jnp.float32),
    )


def kernel(x_feats, cond_feats, nbr_idx, nbr_mask,
           W1a, b1a, W1b, b1b, W2a, b2a, W2b, b2b,
           Wq1, bq1, Wq2, bq2, Wq3, bq3, Wq4, bq4):
    n, N = x_feats.shape
    NQ = cond_feats.shape[1]
    MP = 10240
    assert n <= MP - 1

    xp = jnp.zeros((MP, N), jnp.float32).at[:n].set(x_feats)
    cp = jnp.zeros((MP, NQ), jnp.float32).at[:n].set(cond_feats)
    safe = jnp.where(nbr_mask, nbr_idx, n).astype(jnp.int32)
    idxflat = jnp.full((MP, 27), n, jnp.int32).at[:n].set(safe.T).reshape(MP * 27)

    w1af = W1a.reshape(27 * N, N)
    w1bf = W1b.reshape(27 * N, N)
    w2af = W2a.reshape(27 * N, N)
    w2bf = W2b.reshape(27 * N, N)
    wq1f = Wq1.reshape(27 * NQ, N // 2)

    def r2(b):
        return b.reshape(1, -1)

    gather2 = _make_sc_gather(MP, (N, NQ))
    gather1 = _make_sc_gather(MP, (N,))
    tc1 = _tc1(MP, n)
    tc_film = _tc_conv(MP, n, "film")
    tc_relu = _tc_conv(MP, n, "relu")
    tc_res = _tc_conv(MP, n, "residual")

    G1, Gq = gather2(xp, cp, idxflat)
    h1, c4 = tc1(G1.reshape(MP, 27 * N), Gq.reshape(MP, 27 * NQ),
                 w1af, r2(b1a), wq1f, r2(bq1), Wq2, r2(bq2),
                 Wq3, r2(bq3), Wq4, r2(bq4))
    G2 = gather1(h1, idxflat)
    feats = tc_film(G2.reshape(MP, 27 * N), w1bf, r2(b1b), c4)
    G3 = gather1(feats, idxflat)
    h2 = tc_relu(G3.reshape(MP, 27 * N), w2af, r2(b2a), h1)
    G4 = gather1(h2, idxflat)
    outp = tc_res(G4.reshape(MP, 27 * N), w2bf, r2(b2b), xp)
    return outp[:n]


# speedup vs baseline: 38.5505x; 17.4302x over previous
import functools

import jax
import jax.numpy as jnp
from jax import lax
from jax.experimental import pallas as pl
from jax.experimental.pallas import tpu as pltpu
from jax.experimental.pallas import tpu_sc as plsc

NB = 8


def _make_sc_gather(MP, dims):
    info = plsc.get_sparse_core_info()
    NC, NS = info.num_cores, info.num_subcores
    NW = NC * NS
    R = (MP * 27) // NW
    CH = 120
    NCH = R // CH
    assert (MP * 27) % NW == 0 and R % CH == 0 and NCH % NB == 0

    mesh = plsc.VectorSubcoreMesh(core_axis_name="c", subcore_axis_name="s")
    out_type = tuple(jax.ShapeDtypeStruct((MP * 27, D), jnp.float32) for D in dims)
    if len(dims) == 1:
        out_type = out_type[0]
    scratch = [pltpu.VMEM((R,), jnp.int32)]
    for D in dims:
        scratch.extend(pltpu.VMEM((CH, D), jnp.float32) for _ in range(NB))
    scratch.extend(pltpu.SemaphoreType.DMA for _ in range(NB))

    @functools.partial(pl.kernel, mesh=mesh, out_type=out_type,
                       scratch_types=tuple(scratch),
                       compiler_params=pltpu.CompilerParams(
                           use_tc_tiling_on_sc=False))
    def gather_kernel(*refs):
        nt = len(dims)
        tables = refs[:nt]
        idx_hbm = refs[nt]
        outs = refs[nt + 1: 2 * nt + 1]
        idx_v = refs[2 * nt + 1]
        allbufs = refs[2 * nt + 2: 2 * nt + 2 + NB * nt]
        sems = refs[2 * nt + 2 + NB * nt:]

        wid = lax.axis_index("s") * NC + lax.axis_index("c")
        base = wid * R
        pltpu.sync_copy(idx_hbm.at[pl.ds(base, R)], idx_v)

        for t in range(nt):
            table = tables[t]
            out = outs[t]
            bufs = allbufs[NB * t: NB * t + NB]

            def start(c, buf, sem):
                pltpu.async_copy(table.at[idx_v.at[pl.ds(c * CH, CH)]], buf, sem)

            def wait(buf, sem):
                pltpu.make_async_copy(table.at[pl.ds(0, CH)], buf, sem).wait()

            def write(c, buf):
                pltpu.sync_copy(buf, out.at[pl.ds(base + c * CH, CH)])

            for j in range(NB):
                start(j, bufs[j], sems[j])

            def body(g, carry):
                for j in range(NB):
                    c = g * NB + j
                    wait(bufs[j], sems[j])
                    write(c, bufs[j])
                    start(c + NB, bufs[j], sems[j])
                return carry

            lax.fori_loop(0, NCH // NB - 1, body, 0)
            for j in range(NB):
                c = (NCH // NB - 1) * NB + j
                wait(bufs[j], sems[j])
                write(c, bufs[j])

    return gather_kernel


def _row_mask(blk_m, n_real):
    def f(x):
        row = pl.program_id(0) * blk_m + lax.broadcasted_iota(jnp.int32, (blk_m, 1), 0)
        return jnp.where(row < n_real, x, 0.0)
    return f


def _tc1(MP, n_real, blk_m=512):
    nblk = MP // blk_m
    maskf = _row_mask(blk_m, n_real)

    def body(g1, gq, w1, b1, wq1, bq1, wq2, bq2, wq3, bq3, wq4, bq4,
             h1_ref, c4_ref):
        h = jnp.dot(g1[...], w1[...], preferred_element_type=jnp.float32) + b1[...]
        h1_ref[...] = maskf(jnp.maximum(h, 0.0))
        c = jnp.dot(gq[...], wq1[...], preferred_element_type=jnp.float32) + bq1[...]
        c = jnp.maximum(c, 0.0)
        c = jnp.maximum(jnp.dot(c, wq2[...], preferred_element_type=jnp.float32) + bq2[...], 0.0)
        c = jnp.maximum(jnp.dot(c, wq3[...], preferred_element_type=jnp.float32) + bq3[...], 0.0)
        c4_ref[...] = jnp.dot(c, wq4[...], preferred_element_type=jnp.float32) + bq4[...]

    return pl.pallas_call(
        body,
        grid=(nblk,),
        in_specs=[
            pl.BlockSpec((blk_m, 27 * 64), lambda i: (i, 0)),
            pl.BlockSpec((blk_m, 27 * 32), lambda i: (i, 0)),
            pl.BlockSpec((27 * 64, 64), lambda i: (0, 0)),
            pl.BlockSpec((1, 64), lambda i: (0, 0)),
            pl.BlockSpec((27 * 32, 32), lambda i: (0, 0)),
            pl.BlockSpec((1, 32), lambda i: (0, 0)),
            pl.BlockSpec((32, 64), lambda i: (0, 0)),
            pl.BlockSpec((1, 64), lambda i: (0, 0)),
            pl.BlockSpec((64, 128), lambda i: (0, 0)),
            pl.BlockSpec((1, 128), lambda i: (0, 0)),
            pl.BlockSpec((128, 128), lambda i: (0, 0)),
            pl.BlockSpec((1, 128), lambda i: (0, 0)),
        ],
        out_specs=[
            pl.BlockSpec((blk_m, 64), lambda i: (i, 0)),
            pl.BlockSpec((blk_m, 128), lambda i: (i, 0)),
        ],
        out_shape=[
            jax.ShapeDtypeStruct((MP, 64), jnp.float32),
            jax.ShapeDtypeStruct((MP, 128), jnp.float32),
        ],
    )


def _tc_conv(MP, n_real, mode, blk_m=512):
    nblk = MP // blk_m
    maskf = _row_mask(blk_m, n_real)

    def body(g, w, b, extra, out_ref):
        h = jnp.dot(g[...], w[...], preferred_element_type=jnp.float32) + b[...]
        if mode == "film":
            e = extra[...]
            h = maskf(h * e[:, :64] + e[:, 64:])
        elif mode == "relu":
            h = maskf(jnp.maximum(h, 0.0))
        else:
            h = h + extra[...]
        out_ref[...] = h

    extra_cols = 128 if mode == "film" else 64
    return pl.pallas_call(
        body,
        grid=(nblk,),
        in_specs=[
            pl.BlockSpec((blk_m, 27 * 64), lambda i: (i, 0)),
            pl.BlockSpec((27 * 64, 64), lambda i: (0, 0)),
            pl.BlockSpec((1, 64), lambda i: (0, 0)),
            pl.BlockSpec((blk_m, extra_cols), lambda i: (i, 0)),
        ],
        out_specs=pl.BlockSpec((blk_m, 64), lambda i: (i, 0)),
        out_shape=jax.ShapeDtypeStruct((MP, 64), jnp.float32),
    )


def kernel(x_feats, cond_feats, nbr_idx, nbr_mask,
           W1a, b1a, W1b, b1b, W2a, b2a, W2b, b2b,
           Wq1, bq1, Wq2, bq2, Wq3, bq3, Wq4, bq4):
    n, N = x_feats.shape
    NQ = cond_feats.shape[1]
    MP = 10240
    assert n <= MP - 1

    xp = jnp.zeros((MP, N), jnp.float32).at[:n].set(x_feats)
    cp = jnp.zeros((MP, NQ), jnp.float32).at[:n].set(cond_feats)
    zspan = MP - n
    ii = jnp.arange(MP, dtype=jnp.int32)[:, None]
    kk = jnp.arange(27, dtype=jnp.int32)[None, :]
    dummy = (n + (ii + 7 * kk) % zspan).astype(jnp.int32)
    idxT = jnp.where(nbr_mask.T, nbr_idx.T.astype(jnp.int32), dummy[:n])
    idxflat = dummy.at[:n].set(idxT).reshape(MP * 27)

    w1af = W1a.reshape(27 * N, N)
    w1bf = W1b.reshape(27 * N, N)
    w2af = W2a.reshape(27 * N, N)
    w2bf = W2b.reshape(27 * N, N)
    wq1f = Wq1.reshape(27 * NQ, N // 2)

    def r2(b):
        return b.reshape(1, -1)

    gather2 = _make_sc_gather(MP, (N, NQ))
    gather1 = _make_sc_gather(MP, (N,))
    tc1 = _tc1(MP, n)
    tc_film = _tc_conv(MP, n, "film")
    tc_relu = _tc_conv(MP, n, "relu")
    tc_res = _tc_conv(MP, n, "residual")

    G1, Gq = gather2(xp, cp, idxflat)
    h1, c4 = tc1(G1.reshape(MP, 27 * N), Gq.reshape(MP, 27 * NQ),
                 w1af, r2(b1a), wq1f, r2(bq1), Wq2, r2(bq2),
                 Wq3, r2(bq3), Wq4, r2(bq4))
    G2 = gather1(h1, idxflat)
    feats = tc_film(G2.reshape(MP, 27 * N), w1bf, r2(b1b), c4)
    G3 = gather1(feats, idxflat)
    h2 = tc_relu(G3.reshape(MP, 27 * N), w2af, r2(b2a), h1)
    G4 = gather1(h2, idxflat)
    outp = tc_res(G4.reshape(MP, 27 * N), w2bf, r2(b2b), xp)
    return outp[:n]
